# Initial kernel scaffold; baseline (speedup 1.0000x reference)
import jax
import jax.numpy as jnp
from jax.experimental import pallas as pl


def kernel(x, Pi, centroids, S, decision_boundaries):
    raise NotImplementedError("write your pallas kernel here")


# speedup vs baseline: 5.0048x; 5.0048x over previous
import functools
import math

import jax
import jax.numpy as jnp
from jax.experimental import pallas as pl
from jax.experimental.pallas import tpu as pltpu

_OUTLIER_RATIO = 0.08


def _mask_kernel(x_ref, mask_ref, sum_ref, mean_ref, var_ref, *, nblocks, n, k):
    p = pl.program_id(0)
    i = pl.program_id(1)
    d = mask_ref.shape[1]

    @pl.when((p == 0) & (i == 0))
    def _():
        sum_ref[...] = jnp.zeros_like(sum_ref)

    @pl.when(p == 0)
    def _():
        sum_ref[...] += jnp.sum(x_ref[...], axis=0, keepdims=True)

    @pl.when((p == 1) & (i == 0))
    def _():
        mean_ref[...] = sum_ref[...] / n
        var_ref[...] = jnp.zeros_like(var_ref)

    @pl.when(p == 1)
    def _():
        dev = x_ref[...] - mean_ref[...]
        var_ref[...] += jnp.sum(dev * dev, axis=0, keepdims=True)

    @pl.when((p == 1) & (i == nblocks - 1))
    def _():
        var = var_ref[...] / n
        vcol = var.reshape(d, 1)
        jj = jax.lax.broadcasted_iota(jnp.int32, (d, d), 1)
        ii = jax.lax.broadcasted_iota(jnp.int32, (d, d), 0)
        before = (var > vcol) | ((var == vcol) & (jj < ii))
        rank = jnp.sum(before.astype(jnp.int32), axis=1, keepdims=True)
        mask_col = jnp.where(rank < k, 0.0, 1.0)
        mask_ref[...] = mask_col.reshape(1, d)


def _f16_round(v):
    u = jax.lax.bitcast_convert_type(v, jnp.uint32)
    sign = u & jnp.uint32(0x80000000)
    a = u & jnp.uint32(0x7FFFFFFF)
    lsb = (a >> 13) & jnp.uint32(1)
    rn = (a + jnp.uint32(0xFFF) + lsb) & jnp.uint32(0xFFFFE000)
    normal = jax.lax.bitcast_convert_type(rn | sign, jnp.float32)
    av = jnp.abs(v)
    sub_mag = (av + 0.5) - 0.5
    sub = jnp.where(v < 0, -sub_mag, sub_mag)
    return jnp.where(a >= jnp.uint32(0x38800000), normal, sub)


def _main_kernel(x_ref, pi_ref, s_ref, mask_ref, cent_ref, bound_ref, out_ref,
                 *, scale):
    xb = x_ref[...]
    norms = jnp.sqrt(jnp.sum(xb * xb, axis=1, keepdims=True))
    xu = xb / (norms + 1e-10)
    m = mask_ref[...]
    xr = xu * m
    rn = jnp.sqrt(jnp.sum(xr * xr, axis=1, keepdims=True))
    xru = xr / (rn + 1e-10)
    pi = pi_ref[...]
    y = jax.lax.dot_general(xru, pi, (((1,), (1,)), ((), ())),
                            preferred_element_type=jnp.float32)
    b0, b1, b2 = bound_ref[0], bound_ref[1], bound_ref[2]
    c0, c1, c2, c3 = cent_ref[0], cent_ref[1], cent_ref[2], cent_ref[3]
    yh = jnp.where(y > b1,
                   jnp.where(y > b2, c3, c2),
                   jnp.where(y > b0, c1, c0))
    xm = jax.lax.dot_general(yh, pi, (((1,), (0,)), ((), ())),
                             preferred_element_type=jnp.float32) * rn
    r = xr - xm
    s = s_ref[...]
    proj = jax.lax.dot_general(r, s, (((1,), (1,)), ((), ())),
                               preferred_element_type=jnp.float32)
    sg = jnp.where(proj > 0, 1.0, -1.0)
    resn = jnp.sqrt(jnp.sum(r * r, axis=1, keepdims=True))
    rh = jax.lax.dot_general(sg, s, (((1,), (0,)), ((), ())),
                             preferred_element_type=jnp.float32)
    xh = xm + rh * (scale * resn)
    pt = _f16_round(xu)
    out_ref[...] = jnp.where(m > 0.5, xh, pt) * norms


def kernel(x, Pi, centroids, S, decision_boundaries):
    n, d = x.shape
    k = max(1, int(d * _OUTLIER_RATIO))
    scale = math.sqrt(math.pi / 2.0) / d

    blk_stats = 512
    nblocks = n // blk_stats
    mask = pl.pallas_call(
        functools.partial(_mask_kernel, nblocks=nblocks, n=float(n), k=k),
        grid=(2, nblocks),
        in_specs=[pl.BlockSpec((blk_stats, d), lambda p, i: (i, 0))],
        out_specs=pl.BlockSpec((1, d), lambda p, i: (0, 0)),
        out_shape=jax.ShapeDtypeStruct((1, d), jnp.float32),
        scratch_shapes=[pltpu.VMEM((1, d), jnp.float32),
                        pltpu.VMEM((1, d), jnp.float32),
                        pltpu.VMEM((1, d), jnp.float32)],
    )(x)

    blk = 512
    out = pl.pallas_call(
        functools.partial(_main_kernel, scale=scale),
        grid=(n // blk,),
        in_specs=[
            pl.BlockSpec((blk, d), lambda i: (i, 0)),
            pl.BlockSpec((d, d), lambda i: (0, 0)),
            pl.BlockSpec((d, d), lambda i: (0, 0)),
            pl.BlockSpec((1, d), lambda i: (0, 0)),
            pl.BlockSpec(memory_space=pltpu.SMEM),
            pl.BlockSpec(memory_space=pltpu.SMEM),
        ],
        out_specs=pl.BlockSpec((blk, d), lambda i: (i, 0)),
        out_shape=jax.ShapeDtypeStruct((n, d), jnp.float32),
    )(x, Pi, S, mask, centroids, decision_boundaries)
    return out


# speedup vs baseline: 6.3578x; 1.2703x over previous
import functools
import math

import jax
import jax.numpy as jnp
from jax.experimental import pallas as pl
from jax.experimental.pallas import tpu as pltpu

_OUTLIER_RATIO = 0.08


def _f16_round(v):
    u = jax.lax.bitcast_convert_type(v, jnp.uint32)
    sign = u & jnp.uint32(0x80000000)
    a = u & jnp.uint32(0x7FFFFFFF)
    lsb = (a >> 13) & jnp.uint32(1)
    rn = (a + jnp.uint32(0xFFF) + lsb) & jnp.uint32(0xFFFFE000)
    normal = jax.lax.bitcast_convert_type(rn | sign, jnp.float32)
    av = jnp.abs(v)
    sub_mag = (av + 0.5) - 0.5
    sub = jnp.where(v < 0, -sub_mag, sub_mag)
    return jnp.where(a >= jnp.uint32(0x38800000), normal, sub)


def _fused_kernel(x_ref, pi_ref, s_ref, cent_ref, bound_ref, out_ref,
                  sum_ref, sumsq_ref, mask_ref, *, nblocks, n, k, scale):
    p = pl.program_id(0)
    i = pl.program_id(1)
    d = out_ref.shape[1]

    @pl.when(p == 0)
    def _stats():
        xb = x_ref[...]

        @pl.when(i == 0)
        def _():
            sum_ref[...] = jnp.zeros_like(sum_ref)
            sumsq_ref[...] = jnp.zeros_like(sumsq_ref)

        sum_ref[...] += jnp.sum(xb, axis=0, keepdims=True)
        sumsq_ref[...] += jnp.sum(xb * xb, axis=0, keepdims=True)

        @pl.when(i == nblocks - 1)
        def _():
            mean = sum_ref[...] / n
            var = sumsq_ref[...] / n - mean * mean
            vcol = var.reshape(d, 1)
            jj = jax.lax.broadcasted_iota(jnp.int32, (d, d), 1)
            ii = jax.lax.broadcasted_iota(jnp.int32, (d, d), 0)
            before = (var > vcol) | ((var == vcol) & (jj < ii))
            rank = jnp.sum(before.astype(jnp.int32), axis=1, keepdims=True)
            mask_col = jnp.where(rank < k, 0.0, 1.0)
            mask_ref[...] = mask_col.reshape(1, d)

    @pl.when(p == 1)
    def _main():
        xb = x_ref[...]
        norms = jnp.sqrt(jnp.sum(xb * xb, axis=1, keepdims=True))
        xu = xb / (norms + 1e-10)
        m = mask_ref[...]
        xr = xu * m
        rn = jnp.sqrt(jnp.sum(xr * xr, axis=1, keepdims=True))
        xru = xr / (rn + 1e-10)
        pi = pi_ref[...]
        y = jax.lax.dot_general(xru, pi, (((1,), (1,)), ((), ())),
                                preferred_element_type=jnp.float32)
        b0, b1, b2 = bound_ref[0], bound_ref[1], bound_ref[2]
        c0, c1, c2, c3 = cent_ref[0], cent_ref[1], cent_ref[2], cent_ref[3]
        yh = jnp.where(y > b1,
                       jnp.where(y > b2, c3, c2),
                       jnp.where(y > b0, c1, c0))
        xm = jax.lax.dot_general(yh, pi, (((1,), (0,)), ((), ())),
                                 preferred_element_type=jnp.float32) * rn
        r = xr - xm
        s = s_ref[...]
        proj = jax.lax.dot_general(r, s, (((1,), (1,)), ((), ())),
                                   preferred_element_type=jnp.float32)
        sg = jnp.where(proj > 0, 1.0, -1.0)
        resn = jnp.sqrt(jnp.sum(r * r, axis=1, keepdims=True))
        rh = jax.lax.dot_general(sg, s, (((1,), (0,)), ((), ())),
                                 preferred_element_type=jnp.float32)
        xh = xm + rh * (scale * resn)
        pt = _f16_round(xu)
        out_ref[...] = jnp.where(m > 0.5, xh, pt) * norms


def kernel(x, Pi, centroids, S, decision_boundaries):
    n, d = x.shape
    k = max(1, int(d * _OUTLIER_RATIO))
    scale = math.sqrt(math.pi / 2.0) / d

    blk = 512
    nblocks = n // blk
    out = pl.pallas_call(
        functools.partial(_fused_kernel, nblocks=nblocks, n=float(n), k=k,
                          scale=scale),
        grid=(2, nblocks),
        in_specs=[
            pl.BlockSpec((blk, d), lambda p, i: (i, 0)),
            pl.BlockSpec((d, d), lambda p, i: (0, 0)),
            pl.BlockSpec((d, d), lambda p, i: (0, 0)),
            pl.BlockSpec(memory_space=pltpu.SMEM),
            pl.BlockSpec(memory_space=pltpu.SMEM),
        ],
        out_specs=pl.BlockSpec((blk, d), lambda p, i: (p * i, 0)),
        out_shape=jax.ShapeDtypeStruct((n, d), jnp.float32),
        scratch_shapes=[pltpu.VMEM((1, d), jnp.float32),
                        pltpu.VMEM((1, d), jnp.float32),
                        pltpu.VMEM((1, d), jnp.float32)],
    )(x, Pi, S, centroids, decision_boundaries)
    return out
